# baseline (speedup 1.0000x reference)
import functools

import jax
import jax.numpy as jnp
from jax import lax
from jax.experimental import pallas as pl
from jax.experimental.pallas import tpu as pltpu
from jax.experimental.pallas import tpu_sc as plsc

D_MODEL = 64
NW = 32
NBUF = 4


def _embed_body(xp_hbm, table_hbm, out_hbm, idx_v, rows_v, *sems):
    wid = lax.axis_index("s") * 2 + lax.axis_index("c")
    steps = idx_v.shape[0]
    n_cols = out_hbm.shape[1]
    base = wid * steps

    pltpu.sync_copy(xp_hbm.at[pl.ds(base, steps)], idx_v)

    def gather(step, buf):
        return pltpu.async_copy(
            table_hbm.at[idx_v.at[step].at[pl.ds(0, n_cols)]],
            rows_v.at[buf],
            sems[buf],
        )

    for b in range(NBUF):
        gather(b, b)

    def outer(o, carry):
        for b in range(NBUF):
            step = o * NBUF + b
            pltpu.make_async_copy(
                table_hbm.at[idx_v.at[0].at[pl.ds(0, n_cols)]],
                rows_v.at[b],
                sems[b],
            ).wait()
            pltpu.sync_copy(rows_v.at[b], out_hbm.at[base + step])
            nxt = step + NBUF

            @pl.when(nxt < steps)
            def _():
                gather(nxt, b)

        return carry

    lax.fori_loop(0, steps // NBUF, outer, 0)


def kernel(x, table):
    n_rows, n_cols = x.shape
    xi = x.astype(jnp.int32)
    xp = jnp.pad(xi, ((0, 0), (0, 128 - n_cols)))
    steps = n_rows // 2 // NW

    mesh = plsc.VectorSubcoreMesh(core_axis_name="c", subcore_axis_name="s")
    run = functools.partial(
        pl.kernel,
        mesh=mesh,
        compiler_params=pltpu.CompilerParams(use_tc_tiling_on_sc=False),
        out_type=jax.ShapeDtypeStruct(
            (n_rows // 2, n_cols, D_MODEL), jnp.float32
        ),
        scratch_types=[
            pltpu.VMEM((steps, 128), jnp.int32),
            pltpu.VMEM((NBUF, n_cols, D_MODEL), jnp.float32),
        ]
        + [pltpu.SemaphoreType.DMA] * NBUF,
    )(_embed_body)

    half = n_rows // 2
    o1 = run(xp[:half], table)
    o2 = run(xp[half:], table)
    return jnp.concatenate([o1, o2], axis=0)


# speedup vs baseline: 1.0465x; 1.0465x over previous
import functools

import jax
import jax.numpy as jnp
from jax import lax
from jax.experimental import pallas as pl
from jax.experimental.pallas import tpu as pltpu
from jax.experimental.pallas import tpu_sc as plsc

D_MODEL = 64
NW = 32
NBUF = 4


def _embed_body(xp_hbm, table_hbm, out_hbm, idx_v, rows_v, *sems):
    wid = lax.axis_index("s") * 2 + lax.axis_index("c")
    steps = idx_v.shape[0]
    n_cols = out_hbm.shape[1]
    base = wid * steps

    pltpu.sync_copy(xp_hbm.at[pl.ds(base, steps)], idx_v)

    def gather(step, buf):
        return pltpu.async_copy(
            table_hbm.at[idx_v.at[step].at[pl.ds(0, n_cols)]],
            rows_v.at[buf],
            sems[buf],
        )

    for b in range(NBUF):
        gather(b, b)

    def outer(o, carry):
        for b in range(NBUF):
            step = o * NBUF + b
            pltpu.make_async_copy(
                table_hbm.at[idx_v.at[0].at[pl.ds(0, n_cols)]],
                rows_v.at[b],
                sems[b],
            ).wait()
            pltpu.sync_copy(rows_v.at[b], out_hbm.at[base + step])
            nxt = step + NBUF

            @pl.when(nxt < steps)
            def _():
                gather(nxt, b)

        return carry

    lax.fori_loop(0, steps // NBUF, outer, 0)


def kernel(x, table):
    n_rows, n_cols = x.shape
    xi = x.astype(jnp.int32)
    xp = jnp.pad(xi, ((0, 0), (0, 128 - n_cols)))
    steps = n_rows // NW

    mesh = plsc.VectorSubcoreMesh(core_axis_name="c", subcore_axis_name="s")
    run = functools.partial(
        pl.kernel,
        mesh=mesh,
        compiler_params=pltpu.CompilerParams(use_tc_tiling_on_sc=False),
        out_type=jax.ShapeDtypeStruct((n_rows, n_cols, D_MODEL), jnp.float32),
        scratch_types=[
            pltpu.VMEM((steps, 128), jnp.int32),
            pltpu.VMEM((NBUF, n_cols, D_MODEL), jnp.float32),
        ]
        + [pltpu.SemaphoreType.DMA] * NBUF,
    )(_embed_body)

    return run(xp, table)


# speedup vs baseline: 1.0658x; 1.0184x over previous
import functools

import jax
import jax.numpy as jnp
from jax import lax
from jax.experimental import pallas as pl
from jax.experimental.pallas import tpu as pltpu
from jax.experimental.pallas import tpu_sc as plsc

D_MODEL = 64
NW = 32
NBUF = 8


def _embed_body(xp_hbm, table_hbm, out_hbm, idx_v, rows_v, *sems):
    wid = lax.axis_index("s") * 2 + lax.axis_index("c")
    steps = idx_v.shape[0]
    n_cols = out_hbm.shape[1]
    base = wid * steps

    pltpu.sync_copy(xp_hbm.at[pl.ds(base, steps)], idx_v)

    def gather(step, buf):
        return pltpu.async_copy(
            table_hbm.at[idx_v.at[step].at[pl.ds(0, n_cols)]],
            rows_v.at[buf],
            sems[buf],
        )

    for b in range(NBUF):
        gather(b, b)

    def outer(o, carry):
        for b in range(NBUF):
            step = o * NBUF + b
            pltpu.make_async_copy(
                table_hbm.at[idx_v.at[0].at[pl.ds(0, n_cols)]],
                rows_v.at[b],
                sems[b],
            ).wait()
            pltpu.sync_copy(rows_v.at[b], out_hbm.at[base + step])
            nxt = step + NBUF

            @pl.when(nxt < steps)
            def _():
                gather(nxt, b)

        return carry

    lax.fori_loop(0, steps // NBUF, outer, 0)


def kernel(x, table):
    n_rows, n_cols = x.shape
    xi = x.astype(jnp.int32)
    xp = jnp.pad(xi, ((0, 0), (0, 128 - n_cols)))
    steps = n_rows // NW

    mesh = plsc.VectorSubcoreMesh(core_axis_name="c", subcore_axis_name="s")
    run = functools.partial(
        pl.kernel,
        mesh=mesh,
        compiler_params=pltpu.CompilerParams(use_tc_tiling_on_sc=False),
        out_type=jax.ShapeDtypeStruct((n_rows, n_cols, D_MODEL), jnp.float32),
        scratch_types=[
            pltpu.VMEM((steps, 128), jnp.int32),
            pltpu.VMEM((NBUF, n_cols, D_MODEL), jnp.float32),
        ]
        + [pltpu.SemaphoreType.DMA] * NBUF,
    )(_embed_body)

    return run(xp, table)
